# baseline (speedup 1.0000x reference)
import functools

import jax
import jax.numpy as jnp
from jax import lax
from jax.experimental import pallas as pl
from jax.experimental.pallas import tpu as pltpu
from jax.experimental.pallas import tpu_sc as plsc


def _matmul_split(x, W0, block_rows=1000):
    N, F = x.shape
    H = W0.shape[1]
    Hh = H // 2
    grid = N // block_rows

    def body(x_ref, w_ref, oa_ref, ob_ref):
        r = jnp.dot(x_ref[...], w_ref[...], preferred_element_type=jnp.float32)
        oa_ref[...] = r[:, :Hh]
        ob_ref[...] = r[:, Hh:]

    return pl.pallas_call(
        body,
        grid=(grid,),
        in_specs=[
            pl.BlockSpec((block_rows, F), lambda i: (i, 0)),
            pl.BlockSpec((F, H), lambda i: (0, 0)),
        ],
        out_specs=[
            pl.BlockSpec((block_rows, Hh), lambda i: (i, 0)),
            pl.BlockSpec((block_rows, Hh), lambda i: (i, 0)),
        ],
        out_shape=[
            jax.ShapeDtypeStruct((N, Hh), jnp.float32),
            jax.ShapeDtypeStruct((N, Hh), jnp.float32),
        ],
    )(x, W0)


def _segsum(table_a, table_b, src1, dst1, ew1, relu, B=40):
    N, H = table_a.shape
    E = ew1.shape[0]
    NS = 16
    EPT = E // NS
    NB = EPT // B
    RC = 40
    NCHUNK = N // RC
    KMAX = (NCHUNK + NS - 1) // NS
    HV = H // 16
    NPAIR = NB // 2
    GROUPS = [(o, 0) for o in range(0, B - 15, 16)]
    if B % 16:
        GROUPS.append((B - 16, 16 - B % 16))

    mesh = plsc.VectorSubcoreMesh(core_axis_name="c", subcore_axis_name="s")

    @functools.partial(
        pl.kernel,
        out_type=[
            jax.ShapeDtypeStruct((N, H), jnp.float32),
            jax.ShapeDtypeStruct((N, H), jnp.float32),
        ],
        mesh=mesh,
        scratch_types=[
            pltpu.VMEM((EPT,), jnp.int32),
            pltpu.VMEM((EPT,), jnp.float32),
            pltpu.VMEM((B, H), jnp.float32),
            pltpu.VMEM((B, H), jnp.float32),
            pltpu.VMEM((B,), jnp.int32),
            pltpu.VMEM((B,), jnp.int32),
            pltpu.VMEM_SHARED((N, H), jnp.float32),
            pltpu.SemaphoreType.DMA,
            pltpu.SemaphoreType.DMA,
            pltpu.SemaphoreType.DMA,
            pltpu.SemaphoreType.DMA,
        ],
    )
    def seg_kernel(ta, tb, src_r, dst_r, ew_r, oa, ob,
                   sidx_v, ew_v, rows0_v, rows1_v, didx0_v, didx1_v,
                   acc, sem0, sem1, semd0, semd1):
        c = lax.axis_index("c")
        s = lax.axis_index("s")

        pltpu.sync_copy(src_r.at[pl.ds(s * EPT, EPT)], sidx_v)
        pltpu.sync_copy(ew_r.at[pl.ds(s * EPT, EPT)], ew_v)

        zvec = jnp.zeros((16,), jnp.float32)

        def zero_body(r, carry):
            for j in range(HV):
                rows0_v[r, pl.ds(j * 16, 16)] = zvec
            return carry

        lax.fori_loop(0, RC, zero_body, 0)
        for k in range(KMAX):
            cid = k * NS + s

            @pl.when(cid < NCHUNK)
            def _():
                pltpu.sync_copy(rows0_v.at[pl.ds(0, RC)], acc.at[pl.ds(cid * RC, RC)])
        plsc.subcore_barrier()

        def dcopy(b, buf, sem):
            pltpu.async_copy(dst_r.at[pl.ds(s * EPT + b * B, B)], buf, sem)

        def dcopy_wait(b, buf, sem):
            pltpu.make_async_copy(
                dst_r.at[pl.ds(s * EPT + b * B, B)], buf, sem).wait()

        def process(table):
            def weight_and_scatter(b, rows_v, dbuf, semd):
                for off, lo in GROUPS:
                    w16 = ew_v[pl.ds(b * B + off, 16)]
                    for l in range(lo, 16):
                        e = off + l
                        w = w16[l]
                        for j in range(HV):
                            sl = pl.ds(j * 16, 16)
                            rows_v[e, sl] = rows_v[e, sl] * w
                dcopy_wait(b, dbuf, semd)
                pltpu.sync_copy(rows_v, acc.at[dbuf], add=True)

            def gather(b, rows_v, sem):
                pltpu.async_copy(table.at[sidx_v.at[pl.ds(b * B, B)]], rows_v, sem)

            def gather_wait(b, rows_v, sem):
                pltpu.make_async_copy(
                    table.at[sidx_v.at[pl.ds(b * B, B)]], rows_v, sem).wait()

            gather(0, rows0_v, sem0)
            gather(1, rows1_v, sem1)
            dcopy(0, didx0_v, semd0)
            dcopy(1, didx1_v, semd1)

            def pair_body(i, carry):
                b0 = 2 * i
                gather_wait(b0, rows0_v, sem0)
                weight_and_scatter(b0, rows0_v, didx0_v, semd0)

                @pl.when(i < NPAIR - 1)
                def _():
                    gather(b0 + 2, rows0_v, sem0)
                    dcopy(b0 + 2, didx0_v, semd0)

                gather_wait(b0 + 1, rows1_v, sem1)
                weight_and_scatter(b0 + 1, rows1_v, didx1_v, semd1)

                @pl.when(i < NPAIR - 1)
                def _():
                    gather(b0 + 3, rows1_v, sem1)
                    dcopy(b0 + 3, didx1_v, semd1)

                return carry

            lax.fori_loop(0, NPAIR, pair_body, 0)

        @pl.when(c == 0)
        def _():
            process(ta)

        @pl.when(c == 1)
        def _():
            process(tb)

        plsc.subcore_barrier()

        def drain(out):
            for k in range(KMAX):
                cid = k * NS + s

                @pl.when(cid < NCHUNK)
                def _():
                    r0 = cid * RC
                    pltpu.sync_copy(acc.at[pl.ds(r0, RC)], rows0_v.at[pl.ds(0, RC)])
                    if relu:
                        def relu_body(r, carry):
                            for j in range(HV):
                                sl = pl.ds(j * 16, 16)
                                rows0_v[r, sl] = jnp.maximum(rows0_v[r, sl], 0.0)
                            return carry

                        lax.fori_loop(0, RC, relu_body, 0)
                    pltpu.sync_copy(rows0_v.at[pl.ds(0, RC)], out.at[pl.ds(r0, RC)])

        @pl.when(c == 0)
        def _():
            drain(oa)

        @pl.when(c == 1)
        def _():
            drain(ob)

    return seg_kernel(table_a, table_b, src1, dst1, ew1)


def _zcompute(ah_a, ah_b, eps, W1, W2, block_rows=1000):
    N, Hh = ah_a.shape
    H2 = W1.shape[1]
    grid = N // block_rows

    def body(a_ref, b_ref, e_ref, w1_ref, w2_ref, z_ref):
        a = a_ref[...]
        b = b_ref[...]
        zm = (jnp.dot(a, w1_ref[:Hh, :], preferred_element_type=jnp.float32)
              + jnp.dot(b, w1_ref[Hh:, :], preferred_element_type=jnp.float32))
        zl = (jnp.dot(a, w2_ref[:Hh, :], preferred_element_type=jnp.float32)
              + jnp.dot(b, w2_ref[Hh:, :], preferred_element_type=jnp.float32))
        z_ref[...] = zm + e_ref[...] * jnp.exp(zl)

    return pl.pallas_call(
        body,
        grid=(grid,),
        in_specs=[
            pl.BlockSpec((block_rows, Hh), lambda i: (i, 0)),
            pl.BlockSpec((block_rows, Hh), lambda i: (i, 0)),
            pl.BlockSpec((block_rows, H2), lambda i: (i, 0)),
            pl.BlockSpec((2 * Hh, H2), lambda i: (0, 0)),
            pl.BlockSpec((2 * Hh, H2), lambda i: (0, 0)),
        ],
        out_specs=pl.BlockSpec((block_rows, H2), lambda i: (i, 0)),
        out_shape=jax.ShapeDtypeStruct((N, H2), jnp.float32),
    )(ah_a, ah_b, eps, W1, W2)


def _decoder(z, block_rows=200):
    N, H2 = z.shape
    grid = N // block_rows

    def body(zr_ref, zc_ref, o_ref):
        o_ref[...] = lax.dot_general(
            zr_ref[...], zc_ref[...],
            (((1,), (1,)), ((), ())),
            preferred_element_type=jnp.float32,
        )

    return pl.pallas_call(
        body,
        grid=(grid,),
        in_specs=[
            pl.BlockSpec((block_rows, H2), lambda i: (i, 0)),
            pl.BlockSpec((N, H2), lambda i: (0, 0)),
        ],
        out_specs=pl.BlockSpec((block_rows, N), lambda i: (i, 0)),
        out_shape=jax.ShapeDtypeStruct((N, N), jnp.float32),
    )(z, z)


def kernel(x, edge_index, edge_weight, eps, W0, W1, W2):
    src = edge_index[0]
    dst = edge_index[1]
    ew = edge_weight
    sup_a, sup_b = _matmul_split(x, W0)
    h1_a, h1_b = _segsum(sup_a, sup_b, src, dst, ew, relu=True)
    ah_a, ah_b = _segsum(h1_a, h1_b, src, dst, ew, relu=False)
    z = _zcompute(ah_a, ah_b, eps, W1, W2)
    return _decoder(z).reshape(-1)


# speedup vs baseline: 1.0267x; 1.0267x over previous
import functools

import jax
import jax.numpy as jnp
from jax import lax
from jax.experimental import pallas as pl
from jax.experimental.pallas import tpu as pltpu
from jax.experimental.pallas import tpu_sc as plsc


def _matmul_split(x, W0, block_rows=1000):
    N, F = x.shape
    H = W0.shape[1]
    Hh = H // 2
    grid = N // block_rows

    def body(x_ref, w_ref, oa_ref, ob_ref):
        r = jnp.dot(x_ref[...], w_ref[...], preferred_element_type=jnp.float32)
        oa_ref[...] = r[:, :Hh]
        ob_ref[...] = r[:, Hh:]

    return pl.pallas_call(
        body,
        grid=(grid,),
        in_specs=[
            pl.BlockSpec((block_rows, F), lambda i: (i, 0)),
            pl.BlockSpec((F, H), lambda i: (0, 0)),
        ],
        out_specs=[
            pl.BlockSpec((block_rows, Hh), lambda i: (i, 0)),
            pl.BlockSpec((block_rows, Hh), lambda i: (i, 0)),
        ],
        out_shape=[
            jax.ShapeDtypeStruct((N, Hh), jnp.float32),
            jax.ShapeDtypeStruct((N, Hh), jnp.float32),
        ],
    )(x, W0)


def _segsum(table_a, table_b, src1, dst1, ew1, relu, B=80):
    N, H = table_a.shape
    E = ew1.shape[0]
    NS = 16
    EPT = E // NS
    NB = EPT // B
    RC = B
    NCHUNK = N // RC
    KMAX = (NCHUNK + NS - 1) // NS
    HV = H // 16
    NPAIR = NB // 2
    GROUPS = [(o, 0) for o in range(0, B - 15, 16)]
    if B % 16:
        GROUPS.append((B - 16, 16 - B % 16))

    mesh = plsc.VectorSubcoreMesh(core_axis_name="c", subcore_axis_name="s")

    @functools.partial(
        pl.kernel,
        out_type=[
            jax.ShapeDtypeStruct((N, H), jnp.float32),
            jax.ShapeDtypeStruct((N, H), jnp.float32),
        ],
        mesh=mesh,
        scratch_types=[
            pltpu.VMEM((EPT,), jnp.int32),
            pltpu.VMEM((EPT,), jnp.float32),
            pltpu.VMEM((B, H), jnp.float32),
            pltpu.VMEM((B, H), jnp.float32),
            pltpu.VMEM((B,), jnp.int32),
            pltpu.VMEM((B,), jnp.int32),
            pltpu.VMEM_SHARED((N, H), jnp.float32),
            pltpu.SemaphoreType.DMA,
            pltpu.SemaphoreType.DMA,
            pltpu.SemaphoreType.DMA,
            pltpu.SemaphoreType.DMA,
        ],
    )
    def seg_kernel(ta, tb, src_r, dst_r, ew_r, oa, ob,
                   sidx_v, ew_v, rows0_v, rows1_v, didx0_v, didx1_v,
                   acc, sem0, sem1, semd0, semd1):
        c = lax.axis_index("c")
        s = lax.axis_index("s")

        pltpu.sync_copy(src_r.at[pl.ds(s * EPT, EPT)], sidx_v)
        pltpu.sync_copy(ew_r.at[pl.ds(s * EPT, EPT)], ew_v)

        zvec = jnp.zeros((16,), jnp.float32)

        def zero_body(r, carry):
            for j in range(HV):
                rows0_v[r, pl.ds(j * 16, 16)] = zvec
            return carry

        lax.fori_loop(0, RC, zero_body, 0)
        for k in range(KMAX):
            cid = k * NS + s

            @pl.when(cid < NCHUNK)
            def _():
                pltpu.sync_copy(rows0_v.at[pl.ds(0, RC)], acc.at[pl.ds(cid * RC, RC)])
        plsc.subcore_barrier()

        def dcopy(b, buf, sem):
            pltpu.async_copy(dst_r.at[pl.ds(s * EPT + b * B, B)], buf, sem)

        def dcopy_wait(b, buf, sem):
            pltpu.make_async_copy(
                dst_r.at[pl.ds(s * EPT + b * B, B)], buf, sem).wait()

        def process(table):
            def weight_and_scatter(b, rows_v, dbuf, semd):
                for off, lo in GROUPS:
                    w16 = ew_v[pl.ds(b * B + off, 16)]
                    for l in range(lo, 16):
                        e = off + l
                        w = w16[l]
                        for j in range(HV):
                            sl = pl.ds(j * 16, 16)
                            rows_v[e, sl] = rows_v[e, sl] * w
                dcopy_wait(b, dbuf, semd)
                pltpu.sync_copy(rows_v, acc.at[dbuf], add=True)

            def gather(b, rows_v, sem):
                pltpu.async_copy(table.at[sidx_v.at[pl.ds(b * B, B)]], rows_v, sem)

            def gather_wait(b, rows_v, sem):
                pltpu.make_async_copy(
                    table.at[sidx_v.at[pl.ds(b * B, B)]], rows_v, sem).wait()

            gather(0, rows0_v, sem0)
            gather(1, rows1_v, sem1)
            dcopy(0, didx0_v, semd0)
            dcopy(1, didx1_v, semd1)

            def pair_body(i, carry):
                b0 = 2 * i
                gather_wait(b0, rows0_v, sem0)
                weight_and_scatter(b0, rows0_v, didx0_v, semd0)
                gather(b0 + 2, rows0_v, sem0)
                dcopy(b0 + 2, didx0_v, semd0)

                gather_wait(b0 + 1, rows1_v, sem1)
                weight_and_scatter(b0 + 1, rows1_v, didx1_v, semd1)

                @pl.when(i < NPAIR - 1)
                def _():
                    gather(b0 + 3, rows1_v, sem1)
                    dcopy(b0 + 3, didx1_v, semd1)

                return carry

            lax.fori_loop(0, NPAIR, pair_body, 0)
            gather_wait(NB - 1, rows0_v, sem0)
            weight_and_scatter(NB - 1, rows0_v, didx0_v, semd0)

        @pl.when(c == 0)
        def _():
            process(ta)

        @pl.when(c == 1)
        def _():
            process(tb)

        plsc.subcore_barrier()

        def drain(out):
            for k in range(KMAX):
                cid = k * NS + s

                @pl.when(cid < NCHUNK)
                def _():
                    r0 = cid * RC
                    pltpu.sync_copy(acc.at[pl.ds(r0, RC)], rows0_v.at[pl.ds(0, RC)])
                    if relu:
                        def relu_body(r, carry):
                            for j in range(HV):
                                sl = pl.ds(j * 16, 16)
                                rows0_v[r, sl] = jnp.maximum(rows0_v[r, sl], 0.0)
                            return carry

                        lax.fori_loop(0, RC, relu_body, 0)
                    pltpu.sync_copy(rows0_v.at[pl.ds(0, RC)], out.at[pl.ds(r0, RC)])

        @pl.when(c == 0)
        def _():
            drain(oa)

        @pl.when(c == 1)
        def _():
            drain(ob)

    return seg_kernel(table_a, table_b, src1, dst1, ew1)


def _zcompute(ah_a, ah_b, eps, W1, W2, block_rows=1000):
    N, Hh = ah_a.shape
    H2 = W1.shape[1]
    grid = N // block_rows

    def body(a_ref, b_ref, e_ref, w1_ref, w2_ref, z_ref):
        a = a_ref[...]
        b = b_ref[...]
        zm = (jnp.dot(a, w1_ref[:Hh, :], preferred_element_type=jnp.float32)
              + jnp.dot(b, w1_ref[Hh:, :], preferred_element_type=jnp.float32))
        zl = (jnp.dot(a, w2_ref[:Hh, :], preferred_element_type=jnp.float32)
              + jnp.dot(b, w2_ref[Hh:, :], preferred_element_type=jnp.float32))
        z_ref[...] = zm + e_ref[...] * jnp.exp(zl)

    return pl.pallas_call(
        body,
        grid=(grid,),
        in_specs=[
            pl.BlockSpec((block_rows, Hh), lambda i: (i, 0)),
            pl.BlockSpec((block_rows, Hh), lambda i: (i, 0)),
            pl.BlockSpec((block_rows, H2), lambda i: (i, 0)),
            pl.BlockSpec((2 * Hh, H2), lambda i: (0, 0)),
            pl.BlockSpec((2 * Hh, H2), lambda i: (0, 0)),
        ],
        out_specs=pl.BlockSpec((block_rows, H2), lambda i: (i, 0)),
        out_shape=jax.ShapeDtypeStruct((N, H2), jnp.float32),
    )(ah_a, ah_b, eps, W1, W2)


def _decoder(z, block_rows=400):
    N, H2 = z.shape
    grid = N // block_rows

    def body(zr_ref, zc_ref, o_ref):
        o_ref[...] = lax.dot_general(
            zr_ref[...], zc_ref[...],
            (((1,), (1,)), ((), ())),
            preferred_element_type=jnp.float32,
        )

    return pl.pallas_call(
        body,
        grid=(grid,),
        in_specs=[
            pl.BlockSpec((block_rows, H2), lambda i: (i, 0)),
            pl.BlockSpec((N, H2), lambda i: (0, 0)),
        ],
        out_specs=pl.BlockSpec((block_rows, N), lambda i: (i, 0)),
        out_shape=jax.ShapeDtypeStruct((N, N), jnp.float32),
    )(z, z)


def kernel(x, edge_index, edge_weight, eps, W0, W1, W2):
    src = edge_index[0]
    dst = edge_index[1]
    ew = edge_weight
    sup_a, sup_b = _matmul_split(x, W0)
    h1_a, h1_b = _segsum(sup_a, sup_b, src, dst, ew, relu=True)
    ah_a, ah_b = _segsum(h1_a, h1_b, src, dst, ew, relu=False)
    z = _zcompute(ah_a, ah_b, eps, W1, W2)
    return _decoder(z).reshape(-1)
